# baseline (speedup 1.0000x reference)
import numpy as np

import jax
import jax.numpy as jnp
from jax import lax
from jax.experimental import pallas as pl
from jax.experimental.pallas import tpu as pltpu
from jax.experimental.pallas import tpu_sc as plsc

_SHAPE = (2048, 8192)
_K = 167772
_N = _SHAPE[0] * _SHAPE[1]
_R = _N - _K

_BLK = 128

_NW = 32
_PER_W = _N // _NW
_CHUNK = 16384
_NCHUNK = _PER_W // _CHUNK
_NB = 4096


_U32 = np.uint32


def _tf2x32(k0, k1, x0, x1):
    ks0, ks1 = _U32(k0), _U32(k1)
    ks2 = _U32(ks0 ^ ks1 ^ _U32(0x1BD11BDA))
    r1 = (13, 15, 26, 6)
    r2 = (17, 29, 16, 24)
    x0 = (x0 + ks0).astype(_U32)
    x1 = (x1 + ks1).astype(_U32)

    def rounds(a, b, rots):
        for r in rots:
            a = (a + b).astype(_U32)
            b = ((b << _U32(r)) | (b >> _U32(32 - r))).astype(_U32)
            b = a ^ b
        return a, b

    for rots, (ka, kb), c in ((r1, (ks1, ks2), 1), (r2, (ks2, ks0), 2),
                              (r1, (ks0, ks1), 3), (r2, (ks1, ks2), 4),
                              (r1, (ks2, ks0), 5)):
        x0, x1 = rounds(x0, x1, rots)
        x0 = (x0 + ka).astype(_U32)
        x1 = (x1 + kb + _U32(c)).astype(_U32)
    return x0, x1


def _np_uniform(k, n):
    bits0, bits1 = _tf2x32(k[0], k[1], np.zeros(n, _U32),
                           np.arange(n, dtype=_U32))
    bits = bits0 ^ bits1
    return ((bits >> _U32(9)) | _U32(0x3F800000)).view(np.float32) \
        - np.float32(1.0)


def _build_consts():
    b0, b1 = _tf2x32(0, 1, np.zeros(2, _U32), np.arange(2, dtype=_U32))
    k1, k2 = (b0[0], b1[0]), (b0[1], b1[1])
    f32 = np.float32
    u1 = _np_uniform(k1, _N)
    g = (-np.log(-np.log(u1 + f32(1e-8), dtype=f32) + f32(1e-8),
                 dtype=f32)).reshape(_SHAPE)
    u2 = _np_uniform(k2, 2 * _N)
    g2 = (-np.log(-np.log(u2 + f32(1e-20), dtype=f32) + f32(1e-20),
                  dtype=f32)).reshape(_N, 2)
    a = g2[:, 1] > g2[:, 0]
    b = (f32(1.0) + g2[:, 1]) > g2[:, 0]
    code = (a.astype(np.int8) + 2 * b.astype(np.int8)).reshape(_SHAPE)
    return g, code


_G, _CODE = _build_consts()


_SIGN = np.int32(-2147483648)


def _to_key(z):
    u = lax.bitcast_convert_type(z, jnp.int32)
    return jnp.bitwise_xor(
        u, jnp.bitwise_or(lax.shift_right_arithmetic(u, 31), _SIGN))

_ROWS_W = _SHAPE[0] // _NW
_CR = 8
_CC = 1024
_NCH = (_ROWS_W // _CR) * (_SHAPE[1] // _CC)


def _make_hist(shift, masked):
    scratch = [
        pltpu.VMEM((_CR, _CC), jnp.float32),
        pltpu.VMEM((_CR, _CC), jnp.float32),
        pltpu.VMEM((_CR, _CC), jnp.float32),
        pltpu.VMEM((_CR, _CC), jnp.float32),
        pltpu.VMEM((_NB * 16,), jnp.int32),
        pltpu.VMEM((8, 512), jnp.int32),
        pltpu.SemaphoreType.DMA,
        pltpu.SemaphoreType.DMA,
    ]
    if masked:
        scratch.append(pltpu.VMEM((8, 128), jnp.int32))

    def body(*refs):
        if masked:
            m_hbm, g_hbm, b1_hbm, out_hbm, bm0, bg0, bm1, bg1, hist, hred, \
                sem0, sem1, b1v = refs
        else:
            m_hbm, g_hbm, out_hbm, bm0, bg0, bm1, bg1, hist, hred, \
                sem0, sem1 = refs
        wid = lax.axis_index("s") * 2 + lax.axis_index("c")
        lane = lax.iota(jnp.int32, 16)
        ones = jnp.ones((16,), jnp.int32)

        def zero_body(i, c):
            hist[pl.ds(i * 16, 16)] = jnp.zeros((16,), jnp.int32)
            return c

        lax.fori_loop(0, _NB, zero_body, None)

        if masked:
            pltpu.sync_copy(b1_hbm, b1v)
            b1 = b1v[0, pl.ds(0, 16)]

        row0 = wid * _ROWS_W

        def chunk_ref(hbm, c):
            r = row0 + jnp.bitwise_and(lax.shift_right_logical(c, 3) * _CR,
                                       _ROWS_W - 1)
            cc = jnp.bitwise_and(c, 7) * _CC
            return hbm.at[pl.ds(pl.multiple_of(r, _CR), _CR),
                          pl.ds(pl.multiple_of(cc, _CC), _CC)]

        def start(c, bm, bg, sem):
            pltpu.make_async_copy(chunk_ref(m_hbm, c), bm, sem).start()
            pltpu.make_async_copy(chunk_ref(g_hbm, c), bg, sem).start()

        def wait(bm, bg, sem):
            pltpu.make_async_copy(chunk_ref(m_hbm, 0), bm, sem).wait()
            pltpu.make_async_copy(chunk_ref(g_hbm, 0), bg, sem).wait()

        def process(bm, bg):
            for r in range(_CR):
                def ib(j, c2, _r=r):
                    base = j * 256
                    ks = [_to_key(bm[_r, pl.ds(base + u * 16, 16)]
                                  + bg[_r, pl.ds(base + u * 16, 16)])
                          for u in range(16)]
                    idxs = [jnp.bitwise_or(
                        lax.shift_left(jnp.bitwise_and(
                            lax.shift_right_logical(v, shift), _NB - 1), 4),
                        lane) for v in ks]
                    if masked:
                        ms = [lax.shift_right_logical(v, 20) == b1
                              for v in ks]
                        for ix, m in zip(idxs, ms):
                            plsc.addupdate_scatter(hist, [ix], ones, mask=m)
                    else:
                        for ix in idxs:
                            plsc.addupdate_scatter(hist, [ix], ones)
                    return c2

                lax.fori_loop(0, _CC // 256, ib, None)

        start(0, bm0, bg0, sem0)

        def pair(p, c):
            wait(bm0, bg0, sem0)
            start(2 * p + 1, bm1, bg1, sem1)
            process(bm0, bg0)
            wait(bm1, bg1, sem1)
            start(jnp.minimum(2 * p + 2, _NCH - 1), bm0, bg0, sem0)
            process(bm1, bg1)
            return c

        lax.fori_loop(0, _NCH // 2, pair, None)
        wait(bm0, bg0, sem0)

        def red_body(j, c):
            base = lane * 16 + j * 256
            acc = jnp.zeros((16,), jnp.int32)
            for l in range(16):
                acc = acc + plsc.load_gather(hist, [base + l])
            rr = lax.shift_right_logical(j, 5)
            cc = jnp.bitwise_and(j, 31) * 16
            hred[rr, pl.ds(cc, 16)] = acc
            return c

        lax.fori_loop(0, _NB // 16, red_body, None)
        pltpu.sync_copy(hred, out_hbm.at[wid])

    mesh = plsc.VectorSubcoreMesh(core_axis_name="c", subcore_axis_name="s")
    return pl.kernel(
        body,
        out_type=jax.ShapeDtypeStruct((_NW, 8, 512), jnp.int32),
        mesh=mesh,
        scratch_types=scratch,
        compiler_params=pltpu.CompilerParams(
            needs_layout_passes=False, use_tc_tiling_on_sc=True),
    )


_hist_hi = _make_hist(20, masked=False)
_hist_lo = _make_hist(8, masked=True)


def _select_kernel(t_ref, m_ref, g_ref, c_ref, o_ref):
    t = t_ref[0]
    key = _to_key(m_ref[...] + g_ref[...])
    sel = lax.shift_right_logical(key, 8) >= t
    c = c_ref[...].astype(jnp.int32)
    a = jnp.bitwise_and(c, 1).astype(jnp.float32)
    b = jnp.bitwise_and(lax.shift_right_logical(c, 1), 1).astype(jnp.float32)
    o_ref[...] = jnp.where(sel, b, a)


def _select_pass(m, g, code, t24):
    grid = _SHAPE[0] // _BLK
    return pl.pallas_call(
        _select_kernel,
        grid=(grid,),
        in_specs=[
            pl.BlockSpec(memory_space=pltpu.SMEM),
            pl.BlockSpec((_BLK, _SHAPE[1]), lambda i: (i, 0)),
            pl.BlockSpec((_BLK, _SHAPE[1]), lambda i: (i, 0)),
            pl.BlockSpec((_BLK, _SHAPE[1]), lambda i: (i, 0)),
        ],
        out_specs=pl.BlockSpec((_BLK, _SHAPE[1]), lambda i: (i, 0)),
        out_shape=jax.ShapeDtypeStruct(_SHAPE, jnp.float32),
    )(t24, m, g, code)


def kernel(mask_logits):
    g = _G
    h1 = jnp.sum(_hist_hi(mask_logits, g).reshape(_NW, _NB), axis=0)
    incl = jnp.cumsum(h1)
    b1 = jnp.argmax(incl > _R).astype(jnp.int32)
    r1 = (_R - (incl[b1] - h1[b1])).astype(jnp.int32)
    h2 = jnp.sum(_hist_lo(mask_logits, g, jnp.full((8, 128), b1, jnp.int32))
                 .reshape(_NW, _NB), axis=0)
    c1 = jnp.argmax(jnp.cumsum(h2) > r1).astype(jnp.int32)
    t24 = ((b1 << 12) | c1).reshape(1)
    return _select_pass(mask_logits, g, _CODE, t24)


# speedup vs baseline: 1.4663x; 1.4663x over previous
import numpy as np

import jax
import jax.numpy as jnp
from jax import lax
from jax.experimental import pallas as pl
from jax.experimental.pallas import tpu as pltpu
from jax.experimental.pallas import tpu_sc as plsc

_SHAPE = (2048, 8192)
_K = 167772
_N = _SHAPE[0] * _SHAPE[1]
_R = _N - _K

_BLK = 128

_NW = 32
_PER_W = _N // _NW
_CHUNK = 16384
_NCHUNK = _PER_W // _CHUNK
_NB = 4096


_U32 = np.uint32


def _tf2x32(k0, k1, x0, x1):
    ks0, ks1 = _U32(k0), _U32(k1)
    ks2 = _U32(ks0 ^ ks1 ^ _U32(0x1BD11BDA))
    r1 = (13, 15, 26, 6)
    r2 = (17, 29, 16, 24)
    x0 = (x0 + ks0).astype(_U32)
    x1 = (x1 + ks1).astype(_U32)

    def rounds(a, b, rots):
        for r in rots:
            a = (a + b).astype(_U32)
            b = ((b << _U32(r)) | (b >> _U32(32 - r))).astype(_U32)
            b = a ^ b
        return a, b

    for rots, (ka, kb), c in ((r1, (ks1, ks2), 1), (r2, (ks2, ks0), 2),
                              (r1, (ks0, ks1), 3), (r2, (ks1, ks2), 4),
                              (r1, (ks2, ks0), 5)):
        x0, x1 = rounds(x0, x1, rots)
        x0 = (x0 + ka).astype(_U32)
        x1 = (x1 + kb + _U32(c)).astype(_U32)
    return x0, x1


def _np_uniform(k, n):
    bits0, bits1 = _tf2x32(k[0], k[1], np.zeros(n, _U32),
                           np.arange(n, dtype=_U32))
    bits = bits0 ^ bits1
    return ((bits >> _U32(9)) | _U32(0x3F800000)).view(np.float32) \
        - np.float32(1.0)


def _build_consts():
    b0, b1 = _tf2x32(0, 1, np.zeros(2, _U32), np.arange(2, dtype=_U32))
    k1, k2 = (b0[0], b1[0]), (b0[1], b1[1])
    f32 = np.float32
    u1 = _np_uniform(k1, _N)
    g = (-np.log(-np.log(u1 + f32(1e-8), dtype=f32) + f32(1e-8),
                 dtype=f32)).reshape(_SHAPE)
    u2 = _np_uniform(k2, 2 * _N)
    g2 = (-np.log(-np.log(u2 + f32(1e-20), dtype=f32) + f32(1e-20),
                  dtype=f32)).reshape(_N, 2)
    a = g2[:, 1] > g2[:, 0]
    b = (f32(1.0) + g2[:, 1]) > g2[:, 0]
    code = (a.astype(np.int8) + 2 * b.astype(np.int8)).reshape(_SHAPE)
    return g, code


_G, _CODE = _build_consts()


def _key_kernel(m_ref, g_ref, o_ref):
    z = m_ref[...] + g_ref[...]
    u = lax.bitcast_convert_type(z, jnp.uint32)
    uk = jnp.where(u >= jnp.uint32(0x80000000), jnp.invert(u),
                   u | jnp.uint32(0x80000000))
    o_ref[...] = lax.bitcast_convert_type(uk, jnp.int32)


def _key_pass(m, g):
    grid = _SHAPE[0] // _BLK
    return pl.pallas_call(
        _key_kernel,
        grid=(grid,),
        in_specs=[pl.BlockSpec((_BLK, _SHAPE[1]), lambda i: (i, 0))] * 2,
        out_specs=pl.BlockSpec((_BLK, _SHAPE[1]), lambda i: (i, 0)),
        out_shape=jax.ShapeDtypeStruct(_SHAPE, jnp.int32),
    )(m, g)


_ROWS_W = _SHAPE[0] // _NW
_CR = 8
_CC = 1024
_NCH = (_ROWS_W // _CR) * (_SHAPE[1] // _CC)


def _make_hist(shift, masked):
    scratch = [
        pltpu.VMEM((_CR, _CC), jnp.int32),
        pltpu.VMEM((_CR, _CC), jnp.int32),
        pltpu.VMEM((_NB * 16,), jnp.int32),
        pltpu.VMEM((8, 512), jnp.int32),
        pltpu.SemaphoreType.DMA,
        pltpu.SemaphoreType.DMA,
    ]
    if masked:
        scratch.append(pltpu.VMEM((8, 128), jnp.int32))

    def body(*refs):
        if masked:
            key_hbm, b1_hbm, out_hbm, buf0, buf1, hist, hred, sem0, sem1, \
                b1v = refs
        else:
            key_hbm, out_hbm, buf0, buf1, hist, hred, sem0, sem1 = refs
        wid = lax.axis_index("s") * 2 + lax.axis_index("c")
        lane = lax.iota(jnp.int32, 16)
        ones = jnp.ones((16,), jnp.int32)

        def zero_body(i, c):
            hist[pl.ds(i * 16, 16)] = jnp.zeros((16,), jnp.int32)
            return c

        lax.fori_loop(0, _NB, zero_body, None)

        if masked:
            pltpu.sync_copy(b1_hbm, b1v)
            b1 = b1v[0, pl.ds(0, 16)]

        row0 = wid * _ROWS_W

        def chunk_ref(c):
            r = row0 + jnp.bitwise_and(lax.shift_right_logical(c, 3) * _CR,
                                       _ROWS_W - 1)
            cc = jnp.bitwise_and(c, 7) * _CC
            return key_hbm.at[pl.ds(pl.multiple_of(r, _CR), _CR),
                              pl.ds(pl.multiple_of(cc, _CC), _CC)]

        def start(c, buf, sem):
            pltpu.make_async_copy(chunk_ref(c), buf, sem).start()

        def wait(buf, sem):
            pltpu.make_async_copy(chunk_ref(0), buf, sem).wait()

        def process(buf):
            for r in range(_CR):
                def ib(j, c2, _r=r):
                    base = j * 256
                    vs = [buf[_r, pl.ds(base + u * 16, 16)]
                          for u in range(16)]
                    idxs = [jnp.bitwise_or(
                        lax.shift_left(jnp.bitwise_and(
                            lax.shift_right_logical(v, shift), _NB - 1), 4),
                        lane) for v in vs]
                    if masked:
                        ms = [lax.shift_right_logical(v, 20) == b1
                              for v in vs]
                        for ix, m in zip(idxs, ms):
                            plsc.addupdate_scatter(hist, [ix], ones, mask=m)
                    else:
                        for ix in idxs:
                            plsc.addupdate_scatter(hist, [ix], ones)
                    return c2

                lax.fori_loop(0, _CC // 256, ib, None)

        start(0, buf0, sem0)

        def pair(p, c):
            wait(buf0, sem0)
            start(2 * p + 1, buf1, sem1)
            process(buf0)
            wait(buf1, sem1)
            start(jnp.minimum(2 * p + 2, _NCH - 1), buf0, sem0)
            process(buf1)
            return c

        lax.fori_loop(0, _NCH // 2, pair, None)
        wait(buf0, sem0)

        def red_body(j, c):
            base = lane * 16 + j * 256
            acc = jnp.zeros((16,), jnp.int32)
            for l in range(16):
                acc = acc + plsc.load_gather(hist, [base + l])
            rr = lax.shift_right_logical(j, 5)
            cc = jnp.bitwise_and(j, 31) * 16
            hred[rr, pl.ds(cc, 16)] = acc
            return c

        lax.fori_loop(0, _NB // 16, red_body, None)
        pltpu.sync_copy(hred, out_hbm.at[wid])

    mesh = plsc.VectorSubcoreMesh(core_axis_name="c", subcore_axis_name="s")
    return pl.kernel(
        body,
        out_type=jax.ShapeDtypeStruct((_NW, 8, 512), jnp.int32),
        mesh=mesh,
        scratch_types=scratch,
        compiler_params=pltpu.CompilerParams(
            needs_layout_passes=False, use_tc_tiling_on_sc=True),
    )


_hist_hi = _make_hist(20, masked=False)
_hist_lo = _make_hist(8, masked=True)


def _select_kernel(t_ref, k_ref, c_ref, o_ref):
    t = t_ref[0]
    sel = lax.shift_right_logical(k_ref[...], 8) >= t
    c = c_ref[...].astype(jnp.int32)
    a = jnp.bitwise_and(c, 1).astype(jnp.float32)
    b = jnp.bitwise_and(lax.shift_right_logical(c, 1), 1).astype(jnp.float32)
    o_ref[...] = jnp.where(sel, b, a)


def _select_pass(key, code, t24):
    grid = _SHAPE[0] // _BLK
    return pl.pallas_call(
        _select_kernel,
        grid=(grid,),
        in_specs=[
            pl.BlockSpec(memory_space=pltpu.SMEM),
            pl.BlockSpec((_BLK, _SHAPE[1]), lambda i: (i, 0)),
            pl.BlockSpec((_BLK, _SHAPE[1]), lambda i: (i, 0)),
        ],
        out_specs=pl.BlockSpec((_BLK, _SHAPE[1]), lambda i: (i, 0)),
        out_shape=jax.ShapeDtypeStruct(_SHAPE, jnp.float32),
    )(t24, key, code)


def kernel(mask_logits):
    key = _key_pass(mask_logits, _G)
    h1 = jnp.sum(_hist_hi(key).reshape(_NW, _NB), axis=0)
    incl = jnp.cumsum(h1)
    b1 = jnp.argmax(incl > _R).astype(jnp.int32)
    r1 = (_R - (incl[b1] - h1[b1])).astype(jnp.int32)
    h2 = jnp.sum(_hist_lo(key, jnp.full((8, 128), b1, jnp.int32))
                 .reshape(_NW, _NB), axis=0)
    c1 = jnp.argmax(jnp.cumsum(h2) > r1).astype(jnp.int32)
    t24 = ((b1 << 12) | c1).reshape(1)
    return _select_pass(key, _CODE, t24)
